# baseline (speedup 1.0000x reference)
import functools

import jax
import jax.numpy as jnp
from jax import lax
from jax.experimental import pallas as pl
from jax.experimental.pallas import tpu as pltpu
from jax.experimental.pallas import tpu_sc as plsc

N = 50000
CIN = 128
COUT = 64
K = 27
NB = 3

NC, NS = 2, 16
NW = NC * NS
BROW = 128
NBLK = 13
ROWS_W = NBLK * BROW
NPAD = NW * ROWS_W
BM = 512
BMS = 2048
EPS = 1e-5


def _mm_kernel(x_ref, w_ref, out_ref):
    z = jnp.dot(x_ref[...], w_ref[...], preferred_element_type=jnp.float32)
    out_ref[...] = z.astype(jnp.bfloat16)


def _bn_mm_kernel(h_ref, st_ref, g_ref, be_ref, w_ref, out_ref):
    i = pl.program_id(0)
    mean = st_ref[0:1, :] * (1.0 / N)
    var = st_ref[1:2, :] * (1.0 / N) - mean * mean
    scale = g_ref[...] * lax.rsqrt(var + EPS)
    off = be_ref[...] - mean * scale
    h = jnp.maximum(h_ref[...].astype(jnp.float32) * scale + off, 0.0)
    rows = i * BM + lax.broadcasted_iota(jnp.int32, (BM, COUT), 0)
    h = jnp.where(rows < N, h, 0.0)
    z = jnp.dot(h, w_ref[...], preferred_element_type=jnp.float32)
    out_ref[...] = z.astype(jnp.bfloat16)


def _stats_kernel(h_ref, out_ref):
    i = pl.program_id(0)
    hb = h_ref[...].astype(jnp.float32)
    s = jnp.sum(hb, axis=0, keepdims=True)
    s2 = jnp.sum(hb * hb, axis=0, keepdims=True)
    blk = jnp.concatenate([s, s2, jnp.zeros((6, COUT), jnp.float32)], axis=0)

    @pl.when(i == 0)
    def _():
        out_ref[...] = blk

    @pl.when(i > 0)
    def _():
        out_ref[...] += blk


def _final_kernel(h1_ref, h2_ref, h3_ref, st_ref, g_ref, be_ref, out_ref):
    outs = []
    for b, h_ref in enumerate((h1_ref, h2_ref, h3_ref)):
        mean = st_ref[b, 0:1, :] * (1.0 / N)
        var = st_ref[b, 1:2, :] * (1.0 / N) - mean * mean
        scale = g_ref[b] * lax.rsqrt(var + EPS)
        off = be_ref[b] - mean * scale
        outs.append(h_ref[...].astype(jnp.float32) * scale + off)
    y = jnp.concatenate(outs, axis=1)
    out_ref[...] = jnp.maximum(y, 0.0)


def _make_tc_funcs(interpret: bool = False):
    nb = NPAD // BM
    nbs = NPAD // BMS

    def zmm(xp, wflat, cin):
        return pl.pallas_call(
            _mm_kernel,
            grid=(nb,),
            in_specs=[pl.BlockSpec((BM, cin), lambda i: (i, 0)),
                      pl.BlockSpec((cin, K * COUT), lambda i: (0, 0))],
            out_specs=pl.BlockSpec((BM, K * COUT), lambda i: (i, 0)),
            out_shape=jax.ShapeDtypeStruct((NPAD, K * COUT), jnp.bfloat16),
            interpret=interpret,
        )(xp, wflat)

    def bn_zmm(h, st, g, be, wflat):
        return pl.pallas_call(
            _bn_mm_kernel,
            grid=(nb,),
            in_specs=[pl.BlockSpec((BM, COUT), lambda i: (i, 0)),
                      pl.BlockSpec((8, COUT), lambda i: (0, 0)),
                      pl.BlockSpec((1, COUT), lambda i: (0, 0)),
                      pl.BlockSpec((1, COUT), lambda i: (0, 0)),
                      pl.BlockSpec((COUT, K * COUT), lambda i: (0, 0))],
            out_specs=pl.BlockSpec((BM, K * COUT), lambda i: (i, 0)),
            out_shape=jax.ShapeDtypeStruct((NPAD, K * COUT), jnp.bfloat16),
            interpret=interpret,
        )(h, st, g, be, wflat)

    def stats(h):
        return pl.pallas_call(
            _stats_kernel,
            grid=(nbs,),
            in_specs=[pl.BlockSpec((BMS, COUT), lambda i: (i, 0))],
            out_specs=pl.BlockSpec((8, COUT), lambda i: (0, 0)),
            out_shape=jax.ShapeDtypeStruct((8, COUT), jnp.float32),
            interpret=interpret,
        )(h)

    def final(h1, h2, h3, st, g, be):
        return pl.pallas_call(
            _final_kernel,
            grid=(nb,),
            in_specs=[pl.BlockSpec((BM, COUT), lambda i: (i, 0)),
                      pl.BlockSpec((BM, COUT), lambda i: (i, 0)),
                      pl.BlockSpec((BM, COUT), lambda i: (i, 0)),
                      pl.BlockSpec((NB, 8, COUT), lambda i: (0, 0, 0)),
                      pl.BlockSpec((NB, 1, COUT), lambda i: (0, 0, 0)),
                      pl.BlockSpec((NB, 1, COUT), lambda i: (0, 0, 0))],
            out_specs=pl.BlockSpec((BM, NB * COUT), lambda i: (i, 0)),
            out_shape=jax.ShapeDtypeStruct((NPAD, NB * COUT), jnp.float32),
            interpret=interpret,
        )(h1, h2, h3, st, g, be)

    return zmm, bn_zmm, stats, final


def _make_gather():
    mesh = plsc.VectorSubcoreMesh(core_axis_name="c", subcore_axis_name="s",
                                  num_cores=NC, num_subcores=NS)

    @functools.partial(
        pl.kernel,
        out_type=jax.ShapeDtypeStruct((NPAD, COUT), jnp.bfloat16),
        mesh=mesh,
        scratch_types=[
            pltpu.VMEM((K, BROW), jnp.int32),
            pltpu.VMEM((K, BROW), jnp.int32),
            pltpu.VMEM((BROW, COUT), jnp.bfloat16),
            pltpu.VMEM((BROW, COUT), jnp.bfloat16),
            pltpu.SemaphoreType.DMA,
            pltpu.SemaphoreType.DMA,
        ],
        compiler_params=pltpu.CompilerParams(use_tc_tiling_on_sc=False),
    )
    def gather_sum(z_hbm, nbr_hbm, zero_hbm, out_hbm,
                   idx_a, idx_b, acc_a, acc_b, sem_a, sem_b):
        wid = lax.axis_index("s") * NC + lax.axis_index("c")
        base_w = wid * ROWS_W

        def fire(blk, idx_v, acc_v, sem):
            base = base_w + blk * BROW
            pltpu.sync_copy(nbr_hbm.at[:, pl.ds(base, BROW)], idx_v)
            pltpu.sync_copy(zero_hbm, acc_v)

            @pl.loop(0, K)
            def _fire(k):
                pltpu.async_copy(z_hbm.at[idx_v.at[k]], acc_v, sem, add=True)

        def drain_store(blk, idx_v, acc_v, sem):
            @pl.loop(0, K)
            def _drain(k):
                pltpu.make_async_copy(z_hbm.at[idx_v.at[0]], acc_v, sem).wait()

            base = base_w + blk * BROW
            pltpu.sync_copy(acc_v, out_hbm.at[pl.ds(base, BROW)])

        fire(0, idx_a, acc_a, sem_a)

        @pl.loop(0, (NBLK - 1) // 2)
        def _pair(i):
            fire(2 * i + 1, idx_b, acc_b, sem_b)
            drain_store(2 * i, idx_a, acc_a, sem_a)
            fire(2 * i + 2, idx_a, acc_a, sem_a)
            drain_store(2 * i + 1, idx_b, acc_b, sem_b)

        drain_store(NBLK - 1, idx_a, acc_a, sem_a)

    return gather_sum


def _prep(x, params, nbr1, nbr2, nbr3):
    xp = jnp.zeros((NPAD, CIN), jnp.float32).at[:N].set(x)
    nbr = jnp.full((NB, K, NPAD), N, jnp.int32).at[:, :, :N].set(
        jnp.stack([nbr1, nbr2, nbr3]))
    spread = N + (lax.broadcasted_iota(jnp.int32, (NB, K, NPAD), 2)
                  + 120 * lax.broadcasted_iota(jnp.int32, (NB, K, NPAD), 1)
                  ) % (NPAD - N)
    nbr = jnp.where(nbr >= N, spread, nbr)
    nbr = nbr * K + lax.broadcasted_iota(jnp.int32, (NB, K, NPAD), 1)
    w1 = [params['W%d1' % b].transpose(1, 0, 2).reshape(CIN, K * COUT)
          for b in (1, 2, 3)]
    w2 = [params['W%d2' % b].transpose(1, 0, 2).reshape(COUT, K * COUT)
          for b in (1, 2, 3)]
    g1 = [params['g%d1' % b].reshape(1, COUT) for b in (1, 2, 3)]
    be1 = [params['be%d1' % b].reshape(1, COUT) for b in (1, 2, 3)]
    g2 = jnp.stack([params['g%d2' % b].reshape(1, COUT) for b in (1, 2, 3)])
    be2 = jnp.stack([params['be%d2' % b].reshape(1, COUT) for b in (1, 2, 3)])
    zero = jnp.zeros((BROW, COUT), jnp.bfloat16)
    return xp, nbr, w1, w2, g1, be1, g2, be2, zero


@functools.lru_cache(maxsize=1)
def _get_pipeline():
    zmm, bn_zmm, stats, final = _make_tc_funcs()
    gather_sum = _make_gather()

    def pipeline(x, params, coords, nbr1, nbr2, nbr3):
        xp, nbr, w1, w2, g1, be1, g2, be2, zero = _prep(
            x, params, nbr1, nbr2, nbr3)
        h2s, st2s = [], []
        for b in range(NB):
            z1 = zmm(xp, w1[b], CIN).reshape(NPAD * K, COUT)
            h1 = gather_sum(z1, nbr[b], zero)
            st1 = stats(h1)
            z2 = bn_zmm(h1, st1, g1[b], be1[b], w2[b]).reshape(NPAD * K, COUT)
            h2 = gather_sum(z2, nbr[b], zero)
            h2s.append(h2)
            st2s.append(stats(h2))
        y = final(h2s[0], h2s[1], h2s[2], jnp.stack(st2s), g2, be2)
        return y[:N]

    return pipeline


def kernel(x, params, coords, nbr1, nbr2, nbr3):
    return _get_pipeline()(x, params, coords, nbr1, nbr2, nbr3)


# speedup vs baseline: 1.2053x; 1.2053x over previous
import functools

import jax
import jax.numpy as jnp
from jax import lax
from jax.experimental import pallas as pl
from jax.experimental.pallas import tpu as pltpu
from jax.experimental.pallas import tpu_sc as plsc

N = 50000
CIN = 128
COUT = 64
K = 27
NB = 3

NC, NS = 2, 16
NW = NC * NS
BROW = 128
NBLK = 13
ROWS_W = NBLK * BROW
NPAD = NW * ROWS_W
BM = 512
BMS = 2048
EPS = 1e-5


def _mm_kernel(x_ref, w_ref, out_ref):
    out_ref[...] = jnp.dot(x_ref[...], w_ref[...],
                           preferred_element_type=jnp.float32)


def _bn_mm_kernel(h_ref, st_ref, g_ref, be_ref, w_ref, out_ref):
    i = pl.program_id(0)
    mean = st_ref[0:1, :] * (1.0 / N)
    var = st_ref[1:2, :] * (1.0 / N) - mean * mean
    scale = g_ref[...] * lax.rsqrt(var + EPS)
    off = be_ref[...] - mean * scale
    h = jnp.maximum(h_ref[...] * scale + off, 0.0)
    rows = i * BM + lax.broadcasted_iota(jnp.int32, (BM, COUT), 0)
    h = jnp.where(rows < N, h, 0.0)
    out_ref[...] = jnp.dot(h, w_ref[...], preferred_element_type=jnp.float32)


def _stats_kernel(h_ref, out_ref):
    i = pl.program_id(0)
    hb = h_ref[...]
    s = jnp.sum(hb, axis=0, keepdims=True)
    s2 = jnp.sum(hb * hb, axis=0, keepdims=True)
    blk = jnp.concatenate([s, s2, jnp.zeros((6, COUT), jnp.float32)], axis=0)

    @pl.when(i == 0)
    def _():
        out_ref[...] = blk

    @pl.when(i > 0)
    def _():
        out_ref[...] += blk


def _final_kernel(h1_ref, h2_ref, h3_ref, st_ref, g_ref, be_ref, out_ref):
    outs = []
    for b, h_ref in enumerate((h1_ref, h2_ref, h3_ref)):
        mean = st_ref[b, 0:1, :] * (1.0 / N)
        var = st_ref[b, 1:2, :] * (1.0 / N) - mean * mean
        scale = g_ref[b] * lax.rsqrt(var + EPS)
        off = be_ref[b] - mean * scale
        outs.append(h_ref[...] * scale + off)
    y = jnp.concatenate(outs, axis=1)
    out_ref[...] = jnp.maximum(y, 0.0)


def _make_tc_funcs(interpret: bool = False):
    nb = NPAD // BM
    nbs = NPAD // BMS

    def zmm(xp, wflat, cin):
        return pl.pallas_call(
            _mm_kernel,
            grid=(nb,),
            in_specs=[pl.BlockSpec((BM, cin), lambda i: (i, 0)),
                      pl.BlockSpec((cin, K * COUT), lambda i: (0, 0))],
            out_specs=pl.BlockSpec((BM, K * COUT), lambda i: (i, 0)),
            out_shape=jax.ShapeDtypeStruct((NPAD, K * COUT), jnp.float32),
            interpret=interpret,
        )(xp, wflat)

    def bn_zmm(h, st, g, be, wflat):
        return pl.pallas_call(
            _bn_mm_kernel,
            grid=(nb,),
            in_specs=[pl.BlockSpec((BM, COUT), lambda i: (i, 0)),
                      pl.BlockSpec((8, COUT), lambda i: (0, 0)),
                      pl.BlockSpec((1, COUT), lambda i: (0, 0)),
                      pl.BlockSpec((1, COUT), lambda i: (0, 0)),
                      pl.BlockSpec((COUT, K * COUT), lambda i: (0, 0))],
            out_specs=pl.BlockSpec((BM, K * COUT), lambda i: (i, 0)),
            out_shape=jax.ShapeDtypeStruct((NPAD, K * COUT), jnp.float32),
            interpret=interpret,
        )(h, st, g, be, wflat)

    def stats(h):
        return pl.pallas_call(
            _stats_kernel,
            grid=(nbs,),
            in_specs=[pl.BlockSpec((BMS, COUT), lambda i: (i, 0))],
            out_specs=pl.BlockSpec((8, COUT), lambda i: (0, 0)),
            out_shape=jax.ShapeDtypeStruct((8, COUT), jnp.float32),
            interpret=interpret,
        )(h)

    def final(h1, h2, h3, st, g, be):
        return pl.pallas_call(
            _final_kernel,
            grid=(nb,),
            in_specs=[pl.BlockSpec((BM, COUT), lambda i: (i, 0)),
                      pl.BlockSpec((BM, COUT), lambda i: (i, 0)),
                      pl.BlockSpec((BM, COUT), lambda i: (i, 0)),
                      pl.BlockSpec((NB, 8, COUT), lambda i: (0, 0, 0)),
                      pl.BlockSpec((NB, 1, COUT), lambda i: (0, 0, 0)),
                      pl.BlockSpec((NB, 1, COUT), lambda i: (0, 0, 0))],
            out_specs=pl.BlockSpec((BM, NB * COUT), lambda i: (i, 0)),
            out_shape=jax.ShapeDtypeStruct((NPAD, NB * COUT), jnp.float32),
            interpret=interpret,
        )(h1, h2, h3, st, g, be)

    return zmm, bn_zmm, stats, final


def _make_gather():
    mesh = plsc.VectorSubcoreMesh(core_axis_name="c", subcore_axis_name="s",
                                  num_cores=NC, num_subcores=NS)

    @functools.partial(
        pl.kernel,
        out_type=jax.ShapeDtypeStruct((NPAD, COUT), jnp.float32),
        mesh=mesh,
        scratch_types=[
            pltpu.VMEM((K, BROW), jnp.int32),
            pltpu.VMEM((K, BROW), jnp.int32),
            pltpu.VMEM((K, BROW), jnp.int32),
            pltpu.VMEM((BROW, COUT), jnp.float32),
            pltpu.VMEM((BROW, COUT), jnp.float32),
            pltpu.VMEM((BROW, COUT), jnp.float32),
            pltpu.SemaphoreType.DMA,
            pltpu.SemaphoreType.DMA,
            pltpu.SemaphoreType.DMA,
        ],
        compiler_params=pltpu.CompilerParams(use_tc_tiling_on_sc=False),
    )
    def gather_sum(z_hbm, nbr_hbm, zero_hbm, out_hbm,
                   idx_a, idx_b, idx_c, acc_a, acc_b, acc_c,
                   sem_a, sem_b, sem_c):
        wid = lax.axis_index("s") * NC + lax.axis_index("c")
        base_w = wid * ROWS_W

        def fire(blk, idx_v, acc_v, sem):
            base = base_w + blk * BROW
            pltpu.sync_copy(nbr_hbm.at[:, pl.ds(base, BROW)], idx_v)
            pltpu.sync_copy(zero_hbm, acc_v)

            @pl.loop(0, K)
            def _fire(k):
                pltpu.async_copy(z_hbm.at[idx_v.at[k]], acc_v, sem, add=True)

        def drain_store(blk, idx_v, acc_v, sem):
            @pl.loop(0, K)
            def _drain(k):
                pltpu.make_async_copy(z_hbm.at[idx_v.at[0]], acc_v, sem).wait()

            base = base_w + blk * BROW
            pltpu.sync_copy(acc_v, out_hbm.at[pl.ds(base, BROW)])

        fire(0, idx_a, acc_a, sem_a)
        fire(1, idx_b, acc_b, sem_b)
        fire(2, idx_c, acc_c, sem_c)

        @pl.loop(0, (NBLK - 4) // 3)
        def _trip(j):
            drain_store(3 * j, idx_a, acc_a, sem_a)
            fire(3 * j + 3, idx_a, acc_a, sem_a)
            drain_store(3 * j + 1, idx_b, acc_b, sem_b)
            fire(3 * j + 4, idx_b, acc_b, sem_b)
            drain_store(3 * j + 2, idx_c, acc_c, sem_c)
            fire(3 * j + 5, idx_c, acc_c, sem_c)

        drain_store(NBLK - 4, idx_a, acc_a, sem_a)
        fire(NBLK - 1, idx_a, acc_a, sem_a)
        drain_store(NBLK - 3, idx_b, acc_b, sem_b)
        drain_store(NBLK - 2, idx_c, acc_c, sem_c)
        drain_store(NBLK - 1, idx_a, acc_a, sem_a)

    return gather_sum


def _prep(x, params, nbr1, nbr2, nbr3):
    xp = jnp.zeros((NPAD, CIN), jnp.float32).at[:N].set(x)
    nbr = jnp.full((NB, K, NPAD), N, jnp.int32).at[:, :, :N].set(
        jnp.stack([nbr1, nbr2, nbr3]))
    spread = N + (lax.broadcasted_iota(jnp.int32, (NB, K, NPAD), 2)
                  + 120 * lax.broadcasted_iota(jnp.int32, (NB, K, NPAD), 1)
                  ) % (NPAD - N)
    nbr = jnp.where(nbr >= N, spread, nbr)
    nbr = nbr * K + lax.broadcasted_iota(jnp.int32, (NB, K, NPAD), 1)
    w1 = [params['W%d1' % b].transpose(1, 0, 2).reshape(CIN, K * COUT)
          for b in (1, 2, 3)]
    w2 = [params['W%d2' % b].transpose(1, 0, 2).reshape(COUT, K * COUT)
          for b in (1, 2, 3)]
    g1 = [params['g%d1' % b].reshape(1, COUT) for b in (1, 2, 3)]
    be1 = [params['be%d1' % b].reshape(1, COUT) for b in (1, 2, 3)]
    g2 = jnp.stack([params['g%d2' % b].reshape(1, COUT) for b in (1, 2, 3)])
    be2 = jnp.stack([params['be%d2' % b].reshape(1, COUT) for b in (1, 2, 3)])
    zero = jnp.zeros((BROW, COUT), jnp.float32)
    return xp, nbr, w1, w2, g1, be1, g2, be2, zero


@functools.lru_cache(maxsize=1)
def _get_pipeline():
    zmm, bn_zmm, stats, final = _make_tc_funcs()
    gather_sum = _make_gather()

    def pipeline(x, params, coords, nbr1, nbr2, nbr3):
        xp, nbr, w1, w2, g1, be1, g2, be2, zero = _prep(
            x, params, nbr1, nbr2, nbr3)
        h2s, st2s = [], []
        for b in range(NB):
            z1 = zmm(xp, w1[b], CIN).reshape(NPAD * K, COUT)
            h1 = gather_sum(z1, nbr[b], zero)
            st1 = stats(h1)
            z2 = bn_zmm(h1, st1, g1[b], be1[b], w2[b]).reshape(NPAD * K, COUT)
            h2 = gather_sum(z2, nbr[b], zero)
            h2s.append(h2)
            st2s.append(stats(h2))
        y = final(h2s[0], h2s[1], h2s[2], jnp.stack(st2s), g2, be2)
        return y[:N]

    return pipeline


def kernel(x, params, coords, nbr1, nbr2, nbr3):
    return _get_pipeline()(x, params, coords, nbr1, nbr2, nbr3)
